# baseline (speedup 1.0000x reference)
import jax
import jax.numpy as jnp
from jax.experimental import pallas as pl
from jax.experimental.pallas import tpu as pltpu

_OUT_DOM = 64
_DIM_CONT = 128


def _concat_body(x_ref, dv_ref, ttile_ref, out_ref):
    cin = x_ref.shape[1]
    hc = x_ref.shape[2]
    wd = x_ref.shape[3]
    out_ref[0, :cin] = x_ref[0]
    dv = dv_ref[0] + ttile_ref[0:1, :] * 0.0
    out_ref[0, cin:] = jnp.broadcast_to(
        dv.reshape(_OUT_DOM, 1, 1), (_OUT_DOM, hc, wd))


def kernel(x, domain_ids, domain_vectors, W, b, table):
    bsz, cin, h, w = x.shape
    cout = cin + _OUT_DOM
    hc = 32
    nh = h // hc

    dv = jnp.maximum(domain_vectors @ W + b, 0.0) + jnp.take(
        table, domain_ids, axis=0)
    dv3 = dv.reshape(bsz, 1, _OUT_DOM)

    return pl.pallas_call(
        _concat_body,
        grid=(bsz, nh),
        in_specs=[
            pl.BlockSpec((1, cin, hc, w), lambda i, j: (i, 0, j, 0)),
            pl.BlockSpec((1, 1, _OUT_DOM), lambda i, j: (i, 0, 0)),
            pl.BlockSpec((8, _OUT_DOM), lambda i, j: (0, 0)),
        ],
        out_specs=pl.BlockSpec((1, cout, hc, w), lambda i, j: (i, 0, j, 0)),
        out_shape=jax.ShapeDtypeStruct((bsz, cout, h, w), x.dtype),
    )(x, dv3, table)


# speedup vs baseline: 1.3045x; 1.3045x over previous
import jax
import jax.numpy as jnp
from jax.experimental import pallas as pl
from jax.experimental.pallas import tpu as pltpu

_OUT_DOM = 64
_DIM_CONT = 128


def _body(ids_ref, x_ref, dvec_ref, w_ref, b_ref, trow_ref, out_ref):
    cin = x_ref.shape[1]
    hc = x_ref.shape[2]
    wd = x_ref.shape[3]
    out_ref[0, :cin] = x_ref[0]
    dvv = dvec_ref[0]
    dv = jnp.maximum(
        jnp.dot(dvv, w_ref[...], preferred_element_type=jnp.float32) + b_ref[...],
        0.0,
    )
    dv = dv + trow_ref[0]
    out_ref[0, cin:] = jnp.broadcast_to(
        dv.reshape(_OUT_DOM, 1, 1), (_OUT_DOM, hc, wd))


def kernel(x, domain_ids, domain_vectors, W, b, table):
    bsz, cin, h, w = x.shape
    cout = cin + _OUT_DOM
    hc = 56
    nh = h // hc

    t3 = table.reshape(table.shape[0], 1, _OUT_DOM)
    dvec3 = domain_vectors.reshape(bsz, 1, _DIM_CONT)
    b2 = b.reshape(1, _OUT_DOM)

    return pl.pallas_call(
        _body,
        grid_spec=pltpu.PrefetchScalarGridSpec(
            num_scalar_prefetch=1,
            grid=(bsz, nh),
            in_specs=[
                pl.BlockSpec((1, cin, hc, w), lambda i, j, ids: (i, 0, j, 0)),
                pl.BlockSpec((1, 1, _DIM_CONT), lambda i, j, ids: (i, 0, 0)),
                pl.BlockSpec((_DIM_CONT, _OUT_DOM), lambda i, j, ids: (0, 0)),
                pl.BlockSpec((1, _OUT_DOM), lambda i, j, ids: (0, 0)),
                pl.BlockSpec((1, 1, _OUT_DOM), lambda i, j, ids: (ids[i], 0, 0)),
            ],
            out_specs=pl.BlockSpec((1, cout, hc, w), lambda i, j, ids: (i, 0, j, 0)),
        ),
        out_shape=jax.ShapeDtypeStruct((bsz, cout, h, w), x.dtype),
    )(domain_ids, x, dvec3, W, b2, t3)


# speedup vs baseline: 1.3154x; 1.0083x over previous
import jax
import jax.numpy as jnp
from jax.experimental import pallas as pl
from jax.experimental.pallas import tpu as pltpu

_OUT_DOM = 64
_DIM_CONT = 128


def _body(ids_ref, x_ref, dvec_ref, w_ref, b_ref, trow_ref, out_ref):
    cin = x_ref.shape[1]
    hc = x_ref.shape[2]
    wd = x_ref.shape[3]
    out_ref[0, :cin] = x_ref[0]
    dvv = dvec_ref[0]
    dv = jnp.maximum(
        jnp.dot(dvv, w_ref[...], preferred_element_type=jnp.float32) + b_ref[...],
        0.0,
    )
    dv = dv + trow_ref[0]
    out_ref[0, cin:] = jnp.broadcast_to(
        dv.reshape(_OUT_DOM, 1, 1), (_OUT_DOM, hc, wd))


def kernel(x, domain_ids, domain_vectors, W, b, table):
    bsz, cin, h, w = x.shape
    cout = cin + _OUT_DOM
    hc = 112
    nh = h // hc

    t3 = table.reshape(table.shape[0], 1, _OUT_DOM)
    dvec3 = domain_vectors.reshape(bsz, 1, _DIM_CONT)
    b2 = b.reshape(1, _OUT_DOM)

    return pl.pallas_call(
        _body,
        grid_spec=pltpu.PrefetchScalarGridSpec(
            num_scalar_prefetch=1,
            grid=(bsz, nh),
            in_specs=[
                pl.BlockSpec((1, cin, hc, w), lambda i, j, ids: (i, 0, j, 0)),
                pl.BlockSpec((1, 1, _DIM_CONT), lambda i, j, ids: (i, 0, 0)),
                pl.BlockSpec((_DIM_CONT, _OUT_DOM), lambda i, j, ids: (0, 0)),
                pl.BlockSpec((1, _OUT_DOM), lambda i, j, ids: (0, 0)),
                pl.BlockSpec((1, 1, _OUT_DOM), lambda i, j, ids: (ids[i], 0, 0)),
            ],
            out_specs=pl.BlockSpec((1, cout, hc, w), lambda i, j, ids: (i, 0, j, 0)),
        ),
        out_shape=jax.ShapeDtypeStruct((bsz, cout, h, w), x.dtype),
    )(domain_ids, x, dvec3, W, b2, t3)


# speedup vs baseline: 1.3197x; 1.0032x over previous
import jax
import jax.numpy as jnp
from jax.experimental import pallas as pl
from jax.experimental.pallas import tpu as pltpu

_OUT_DOM = 64
_DIM_CONT = 128


def _body(ids_ref, x_ref, dvec_ref, w_ref, b_ref, trow_ref, out_ref):
    cin = x_ref.shape[1]
    hc = x_ref.shape[2]
    wd = x_ref.shape[3]
    out_ref[0, :cin] = x_ref[0]
    dvv = dvec_ref[0]
    dv = jnp.maximum(
        jnp.dot(dvv, w_ref[...], preferred_element_type=jnp.float32) + b_ref[...],
        0.0,
    )
    dv = dv + trow_ref[0]
    out_ref[0, cin:] = jnp.broadcast_to(
        dv.reshape(_OUT_DOM, 1, 1), (_OUT_DOM, hc, wd))


def kernel(x, domain_ids, domain_vectors, W, b, table):
    bsz, cin, h, w = x.shape
    cout = cin + _OUT_DOM
    hc = 112
    nh = h // hc

    t3 = table.reshape(table.shape[0], 1, _OUT_DOM)
    dvec3 = domain_vectors.reshape(bsz, 1, _DIM_CONT)
    b2 = b.reshape(1, _OUT_DOM)

    return pl.pallas_call(
        _body,
        grid_spec=pltpu.PrefetchScalarGridSpec(
            num_scalar_prefetch=1,
            grid=(bsz, nh),
            in_specs=[
                pl.BlockSpec((1, cin, hc, w), lambda i, j, ids: (i, 0, j, 0)),
                pl.BlockSpec((1, 1, _DIM_CONT), lambda i, j, ids: (i, 0, 0)),
                pl.BlockSpec((_DIM_CONT, _OUT_DOM), lambda i, j, ids: (0, 0)),
                pl.BlockSpec((1, _OUT_DOM), lambda i, j, ids: (0, 0)),
                pl.BlockSpec((1, 1, _OUT_DOM), lambda i, j, ids: (ids[i], 0, 0)),
            ],
            out_specs=pl.BlockSpec((1, cout, hc, w), lambda i, j, ids: (i, 0, j, 0)),
        ),
        out_shape=jax.ShapeDtypeStruct((bsz, cout, h, w), x.dtype),
        compiler_params=pltpu.CompilerParams(
            dimension_semantics=("parallel", "parallel")),
    )(domain_ids, x, dvec3, W, b2, t3)
